# Initial kernel scaffold; baseline (speedup 1.0000x reference)
import functools

import jax
import jax.numpy as jnp
from jax import lax
from jax.experimental import pallas as pl
from jax.experimental.pallas import tpu as pltpu
from jax.experimental.pallas import tpu_sc as plsc

_NUM_SAMPLES = 100000
_NUM_SGS = 24
_FEAT = _NUM_SGS * 7
_BATCH = 16384
_CHUNK = 128


def _make_gather():
    info = plsc.get_sparse_core_info()
    nc, ns = info.num_cores, info.num_subcores
    nw = nc * ns
    b_per_w = _BATCH // nw
    n_chunks = b_per_w // _CHUNK
    mesh = plsc.VectorSubcoreMesh(core_axis_name="c", subcore_axis_name="s")

    @functools.partial(
        pl.kernel,
        mesh=mesh,
        out_type=jax.ShapeDtypeStruct((_BATCH, _FEAT), jnp.float32),
        scratch_types=[
            pltpu.VMEM((n_chunks, _CHUNK), jnp.int32),
            pltpu.VMEM((_CHUNK, _FEAT), jnp.float32),
            pltpu.VMEM((_CHUNK, _FEAT), jnp.float32),
            pltpu.SemaphoreType.DMA,
            pltpu.SemaphoreType.DMA,
        ],
    )
    def gather_kernel(idx_hbm, table_hbm, out_hbm, idx_v, rows0, rows1, sem0, sem1):
        wid = lax.axis_index("s") * nc + lax.axis_index("c")
        base = wid * b_per_w
        pltpu.sync_copy(idx_hbm.at[wid], idx_v)
        rows = (rows0, rows1)
        sems = (sem0, sem1)
        copies = [pltpu.async_copy(table_hbm.at[idx_v.at[0]], rows0, sem0)]
        for c in range(n_chunks):
            if c + 1 < n_chunks:
                copies.append(
                    pltpu.async_copy(
                        table_hbm.at[idx_v.at[c + 1]], rows[(c + 1) % 2],
                        sems[(c + 1) % 2],
                    )
                )
            copies[c].wait()
            pltpu.sync_copy(
                rows[c % 2], out_hbm.at[pl.ds(base + c * _CHUNK, _CHUNK)]
            )

    return gather_kernel, nw, n_chunks


_GATHER, _NW, _NCHUNKS = _make_gather()


def kernel(idxs, sgs):
    idx3 = idxs.astype(jnp.int32).reshape(_NW, _NCHUNKS, _CHUNK)
    table = sgs.reshape(_NUM_SAMPLES, _FEAT)
    out = _GATHER(idx3, table)
    return out.reshape(_BATCH, _NUM_SGS, 7)


# speedup vs baseline: 1.4485x; 1.4485x over previous
import functools

import jax
import jax.numpy as jnp
from jax import lax
from jax.experimental import pallas as pl
from jax.experimental.pallas import tpu as pltpu
from jax.experimental.pallas import tpu_sc as plsc

_NUM_SAMPLES = 100000
_NUM_SGS = 24
_FEAT = _NUM_SGS * 7
_BATCH = 16384
_CHUNK = 128


def _make_gather():
    info = plsc.get_sparse_core_info()
    nc, ns = info.num_cores, info.num_subcores
    nw = nc * ns
    b_per_w = _BATCH // nw
    n_chunks = b_per_w // _CHUNK
    mesh = plsc.VectorSubcoreMesh(core_axis_name="c", subcore_axis_name="s")

    @functools.partial(
        pl.kernel,
        mesh=mesh,
        compiler_params=pltpu.CompilerParams(use_tc_tiling_on_sc=False),
        out_type=jax.ShapeDtypeStruct((_BATCH, _FEAT), jnp.float32),
        scratch_types=[
            pltpu.VMEM((n_chunks, _CHUNK), jnp.int32),
            pltpu.VMEM((_CHUNK, _FEAT), jnp.float32),
            pltpu.VMEM((_CHUNK, _FEAT), jnp.float32),
            pltpu.SemaphoreType.DMA,
            pltpu.SemaphoreType.DMA,
        ],
    )
    def gather_kernel(idx_hbm, table_hbm, out_hbm, idx_v, rows0, rows1, sem0, sem1):
        wid = lax.axis_index("s") * nc + lax.axis_index("c")
        base = wid * b_per_w
        pltpu.sync_copy(idx_hbm.at[wid], idx_v)
        rows = (rows0, rows1)
        sems = (sem0, sem1)
        copies = [pltpu.async_copy(table_hbm.at[idx_v.at[0]], rows0, sem0)]
        for c in range(n_chunks):
            if c + 1 < n_chunks:
                copies.append(
                    pltpu.async_copy(
                        table_hbm.at[idx_v.at[c + 1]], rows[(c + 1) % 2],
                        sems[(c + 1) % 2],
                    )
                )
            copies[c].wait()
            pltpu.sync_copy(
                rows[c % 2], out_hbm.at[pl.ds(base + c * _CHUNK, _CHUNK)]
            )

    return gather_kernel, nw, n_chunks


_GATHER, _NW, _NCHUNKS = _make_gather()


def kernel(idxs, sgs):
    idx3 = idxs.astype(jnp.int32).reshape(_NW, _NCHUNKS, _CHUNK)
    table = sgs.reshape(_NUM_SAMPLES, _FEAT)
    out = _GATHER(idx3, table)
    return out.reshape(_BATCH, _NUM_SGS, 7)
